# baseline (speedup 1.0000x reference)
import functools

import jax
import jax.numpy as jnp
from jax import lax
from jax.experimental import pallas as pl
from jax.experimental.pallas import tpu as pltpu
from jax.experimental.pallas import tpu_sc as plsc

_D = 768
_F = 3072
_E = 8
_S = 2048
_TILE = 256
_TMAX = 24
_CAP = _TMAX * _TILE
_FBLK = 256
_NF = _F // _FBLK
_NW = 32
_TPW = _S // _NW
_LANES = 16


def _gelu(x):
    sqrt_2_pi = 0.7978845608028654
    coef = 0.044715
    x3 = x ** 3
    inner = sqrt_2_pi * (x + coef * x3)
    return 0.5 * x * (1.0 + jnp.tanh(inner))


def _router_kernel(x_ref, w1_ref, b1_ref, lns_ref, lnb_ref, w2_ref, b2_ref,
                   pos0_ref, pos1_ref, w0_ref, w1o_ref, meta_ref):
    x = x_ref[...]
    h = jnp.dot(x, w1_ref[...], preferred_element_type=jnp.float32)
    h = h + b1_ref[...]
    mean = jnp.mean(h, axis=-1, keepdims=True)
    var = jnp.mean(jnp.square(h - mean), axis=-1, keepdims=True)
    h = (h - mean) * lax.rsqrt(var + 1e-6) * lns_ref[...] + lnb_ref[...]
    h = _gelu(h)
    logits = jnp.dot(h, w2_ref[...], preferred_element_type=jnp.float32)
    logits = logits + b2_ref[...]
    lmax = jnp.max(logits, axis=-1, keepdims=True)
    ex = jnp.exp(logits - lmax)
    p = ex / jnp.sum(ex, axis=-1, keepdims=True)

    s, e = p.shape
    ei = lax.broadcasted_iota(jnp.int32, (s, e), 1)
    m1 = jnp.max(p, axis=-1, keepdims=True)
    i1 = jnp.min(jnp.where(p == m1, ei, e), axis=-1, keepdims=True)
    oh1 = ei == i1
    pm = jnp.where(oh1, -jnp.inf, p)
    m2 = jnp.max(pm, axis=-1, keepdims=True)
    i2 = jnp.min(jnp.where(pm == m2, ei, e), axis=-1, keepdims=True)
    oh2 = ei == i2
    denom = m1 + m2
    w0_ref[...] = jnp.broadcast_to(m1 / denom, (s, _LANES))
    w1o_ref[...] = jnp.broadcast_to(m2 / denom, (s, _LANES))

    m01 = jnp.where(oh1 | oh2, 1.0, 0.0)
    l1 = lax.broadcasted_iota(jnp.int32, (128, 128), 0)
    l2 = lax.broadcasted_iota(jnp.int32, (128, 128), 1)
    ltri = jnp.where(l2 <= l1, 1.0, 0.0)
    offs = jnp.zeros((1, e), jnp.float32)
    parts = []
    for c in range(s // 128):
        blk = m01[c * 128:(c + 1) * 128, :]
        cumb = jnp.dot(ltri, blk, preferred_element_type=jnp.float32) + offs
        parts.append(cumb)
        offs = cumb[127:128, :]
    cum = jnp.concatenate(parts, axis=0)

    counts = cum[s - 1:s, :]
    pc = jnp.ceil(counts / _TILE) * _TILE
    e1i = lax.broadcasted_iota(jnp.int32, (e, e), 0)
    e2i = lax.broadcasted_iota(jnp.int32, (e, e), 1)
    slt = jnp.where(e1i < e2i, 1.0, 0.0)
    aoff = jnp.dot(pc, slt, preferred_element_type=jnp.float32)

    g1 = jnp.sum(jnp.where(oh1, aoff + cum, 0.0), axis=-1, keepdims=True) - 1.0
    g2 = jnp.sum(jnp.where(oh2, aoff + cum, 0.0), axis=-1, keepdims=True) - 1.0
    pos0_ref[...] = g1.astype(jnp.int32)
    pos1_ref[...] = g2.astype(jnp.int32)

    endt = (aoff + pc) / _TILE
    eye = e1i == e2i
    end_col = jnp.sum(jnp.where(eye, jnp.broadcast_to(endt, (e, e)), 0.0),
                      axis=-1, keepdims=True)
    ti = lax.broadcasted_iota(jnp.int32, (e, 32), 1).astype(jnp.float32) - 1.0
    full_before = jnp.sum(jnp.where(ti >= end_col, 1.0, 0.0), axis=0,
                          keepdims=True)
    etile = jnp.minimum(full_before, float(e - 1))
    ntile = jnp.sum(pc, axis=-1, keepdims=True) / _TILE
    c32 = lax.broadcasted_iota(jnp.int32, (1, 32), 1)
    meta_ref[...] = jnp.where(c32 == 0, ntile, etile).astype(jnp.int32)


@functools.cache
def _get_dispatch():
    mesh = plsc.VectorSubcoreMesh(core_axis_name="c", subcore_axis_name="s")

    @functools.partial(
        pl.kernel,
        mesh=mesh,
        out_type=jax.ShapeDtypeStruct((_CAP, _D), jnp.float32),
        scratch_types=[
            pltpu.VMEM((_TPW,), jnp.int32),
            pltpu.VMEM((_TPW,), jnp.int32),
            pltpu.VMEM((_TPW, _D), jnp.float32),
            pltpu.SemaphoreType.DMA,
            pltpu.SemaphoreType.DMA,
        ],
    )
    def _dispatch(x_hbm, pos0_hbm, pos1_hbm, xs_hbm, idx0_v, idx1_v, rows_v,
                  sem0, sem1):
        wid = lax.axis_index("s") * 2 + lax.axis_index("c")
        base = wid * _TPW
        pltpu.sync_copy(x_hbm.at[pl.ds(base, _TPW)], rows_v)
        pltpu.sync_copy(pos0_hbm.at[pl.ds(base, _TPW)], idx0_v)
        pltpu.sync_copy(pos1_hbm.at[pl.ds(base, _TPW)], idx1_v)
        c0 = pltpu.async_copy(rows_v, xs_hbm.at[idx0_v], sem0)
        c1 = pltpu.async_copy(rows_v, xs_hbm.at[idx1_v], sem1)
        c0.wait()
        c1.wait()

    return _dispatch


def _expert_kernel(meta_ref, xs_ref, w1a_ref, w1b_ref, b1a_ref, b1b_ref,
                   w2_ref, b2_ref, ys_ref):
    j = pl.program_id(0)
    i = pl.program_id(1)

    @pl.when(i < meta_ref[0])
    def _compute():
        xt = xs_ref[pl.ds(i * _TILE, _TILE), :]
        h1 = jnp.dot(xt, w1a_ref[0], preferred_element_type=jnp.float32)
        h1 = h1 + b1a_ref[0]
        h2 = jnp.dot(xt, w1b_ref[0], preferred_element_type=jnp.float32)
        h2 = h2 + b1b_ref[0]
        g = h1 * _gelu(h2)
        partial = jnp.dot(g, w2_ref[0], preferred_element_type=jnp.float32)

        @pl.when(j == 0)
        def _set():
            ys_ref[pl.ds(i * _TILE, _TILE), :] = partial

        @pl.when(j > 0)
        def _acc():
            ys_ref[pl.ds(i * _TILE, _TILE), :] += partial

        @pl.when(j == _NF - 1)
        def _bias():
            ys_ref[pl.ds(i * _TILE, _TILE), :] += b2_ref[0]


@functools.cache
def _get_combine():
    mesh = plsc.VectorSubcoreMesh(core_axis_name="c", subcore_axis_name="s")

    @functools.partial(
        pl.kernel,
        mesh=mesh,
        out_type=jax.ShapeDtypeStruct((_S, _D), jnp.float32),
        scratch_types=[
            pltpu.VMEM((_TPW,), jnp.int32),
            pltpu.VMEM((_TPW,), jnp.int32),
            pltpu.VMEM((_TPW, _LANES), jnp.float32),
            pltpu.VMEM((_TPW, _LANES), jnp.float32),
            pltpu.VMEM((_TPW, _D), jnp.float32),
            pltpu.VMEM((_TPW, _D), jnp.float32),
            pltpu.SemaphoreType.DMA,
            pltpu.SemaphoreType.DMA,
        ],
    )
    def _combine(ys_hbm, pos0_hbm, pos1_hbm, w0_hbm, w1_hbm, out_hbm,
                 idx0_v, idx1_v, w0_v, w1_v, rows0_v, rows1_v, sem0, sem1):
        wid = lax.axis_index("s") * 2 + lax.axis_index("c")
        base = wid * _TPW
        pltpu.sync_copy(pos0_hbm.at[pl.ds(base, _TPW)], idx0_v)
        pltpu.sync_copy(pos1_hbm.at[pl.ds(base, _TPW)], idx1_v)
        c0 = pltpu.async_copy(ys_hbm.at[idx0_v], rows0_v, sem0)
        c1 = pltpu.async_copy(ys_hbm.at[idx1_v], rows1_v, sem1)
        pltpu.sync_copy(w0_hbm.at[pl.ds(base, _TPW)], w0_v)
        pltpu.sync_copy(w1_hbm.at[pl.ds(base, _TPW)], w1_v)
        c0.wait()
        c1.wait()

        def body(t, carry):
            wv0 = w0_v[t, pl.ds(0, _LANES)]
            wv1 = w1_v[t, pl.ds(0, _LANES)]
            for cc in range(_D // _LANES):
                a = rows0_v[t, pl.ds(cc * _LANES, _LANES)]
                b = rows1_v[t, pl.ds(cc * _LANES, _LANES)]
                rows0_v[t, pl.ds(cc * _LANES, _LANES)] = wv0 * a + wv1 * b
            return carry

        lax.fori_loop(0, _TPW, body, 0)
        pltpu.sync_copy(rows0_v, out_hbm.at[pl.ds(base, _TPW)])

    return _combine


@jax.jit
def kernel(x, r_w1, r_b1, ln_scale, ln_bias, r_w2, r_b2, ew1, eb1, ew2, eb2,
           expert_priors):
    del expert_priors
    b, s, d = x.shape
    x2d = x.reshape(s, d)

    pos0, pos1, w0, w1, meta = pl.pallas_call(
        _router_kernel,
        out_shape=[
            jax.ShapeDtypeStruct((s, 1), jnp.int32),
            jax.ShapeDtypeStruct((s, 1), jnp.int32),
            jax.ShapeDtypeStruct((s, _LANES), jnp.float32),
            jax.ShapeDtypeStruct((s, _LANES), jnp.float32),
            jax.ShapeDtypeStruct((1, 32), jnp.int32),
        ],
    )(x2d, r_w1, r_b1.reshape(1, -1), ln_scale.reshape(1, -1),
      ln_bias.reshape(1, -1), r_w2, r_b2.reshape(1, -1))

    pos0f = pos0.reshape(s)
    pos1f = pos1.reshape(s)
    meta32 = meta.reshape(32)

    xs = _get_dispatch()(x2d, pos0f, pos1f)

    eb1_3d = eb1.reshape(_E, 1, 2 * _F)
    eb2_3d = eb2.reshape(_E, 1, _D)
    grid_spec = pltpu.PrefetchScalarGridSpec(
        num_scalar_prefetch=1,
        grid=(_NF, _TMAX),
        in_specs=[
            pl.BlockSpec((_CAP, _D), lambda j, i, m: (0, 0)),
            pl.BlockSpec((1, _D, _FBLK), lambda j, i, m: (m[1 + i], 0, j)),
            pl.BlockSpec((1, _D, _FBLK), lambda j, i, m: (m[1 + i], 0, j + _NF)),
            pl.BlockSpec((1, 1, _FBLK), lambda j, i, m: (m[1 + i], 0, j)),
            pl.BlockSpec((1, 1, _FBLK), lambda j, i, m: (m[1 + i], 0, j + _NF)),
            pl.BlockSpec((1, _FBLK, _D), lambda j, i, m: (m[1 + i], j, 0)),
            pl.BlockSpec((1, 1, _D), lambda j, i, m: (m[1 + i], 0, 0)),
        ],
        out_specs=pl.BlockSpec((_CAP, _D), lambda j, i, m: (0, 0)),
    )
    ys = pl.pallas_call(
        _expert_kernel,
        grid_spec=grid_spec,
        out_shape=jax.ShapeDtypeStruct((_CAP, _D), jnp.float32),
        compiler_params=pltpu.CompilerParams(
            dimension_semantics=("arbitrary", "arbitrary"),
        ),
    )(meta32, xs, ew1, ew1, eb1_3d, eb1_3d, ew2, eb2_3d)

    out = _get_combine()(ys, pos0f, pos1f, w0, w1)

    return (out.reshape(b, s, d), 0.0)


# speedup vs baseline: 1.3604x; 1.3604x over previous
import functools

import jax
import jax.numpy as jnp
from jax import lax
from jax.experimental import pallas as pl
from jax.experimental.pallas import tpu as pltpu
from jax.experimental.pallas import tpu_sc as plsc

_D = 768
_F = 3072
_E = 8
_S = 2048
_TILE = 128
_TMAX = 40
_CAP = _TMAX * _TILE
_FBLK = 1024
_NF = _F // _FBLK
_NW = 32
_TPW = _S // _NW
_LANES = 16


def _gelu(x):
    sqrt_2_pi = 0.7978845608028654
    coef = 0.044715
    x3 = x ** 3
    inner = sqrt_2_pi * (x + coef * x3)
    return 0.5 * x * (1.0 + jnp.tanh(inner))


def _router_kernel(x_ref, w1_ref, b1_ref, lns_ref, lnb_ref, w2_ref, b2_ref,
                   pos0_ref, pos1_ref, w0_ref, w1o_ref, meta_ref):
    x = x_ref[...]
    h = jnp.dot(x, w1_ref[...], preferred_element_type=jnp.float32)
    h = h + b1_ref[...]
    mean = jnp.mean(h, axis=-1, keepdims=True)
    var = jnp.mean(jnp.square(h - mean), axis=-1, keepdims=True)
    h = (h - mean) * lax.rsqrt(var + 1e-6) * lns_ref[...] + lnb_ref[...]
    h = _gelu(h)
    logits = jnp.dot(h, w2_ref[...], preferred_element_type=jnp.float32)
    logits = logits + b2_ref[...]
    lmax = jnp.max(logits, axis=-1, keepdims=True)
    ex = jnp.exp(logits - lmax)
    p = ex / jnp.sum(ex, axis=-1, keepdims=True)

    s, e = p.shape
    ei = lax.broadcasted_iota(jnp.int32, (s, e), 1)
    m1 = jnp.max(p, axis=-1, keepdims=True)
    i1 = jnp.min(jnp.where(p == m1, ei, e), axis=-1, keepdims=True)
    oh1 = ei == i1
    pm = jnp.where(oh1, -jnp.inf, p)
    m2 = jnp.max(pm, axis=-1, keepdims=True)
    i2 = jnp.min(jnp.where(pm == m2, ei, e), axis=-1, keepdims=True)
    oh2 = ei == i2
    denom = m1 + m2
    w0_ref[...] = jnp.broadcast_to(m1 / denom, (s, _LANES))
    w1o_ref[...] = jnp.broadcast_to(m2 / denom, (s, _LANES))

    m01 = jnp.where(oh1 | oh2, 1.0, 0.0)
    l1 = lax.broadcasted_iota(jnp.int32, (128, 128), 0)
    l2 = lax.broadcasted_iota(jnp.int32, (128, 128), 1)
    ltri = jnp.where(l2 <= l1, 1.0, 0.0)
    offs = jnp.zeros((1, e), jnp.float32)
    parts = []
    for c in range(s // 128):
        blk = m01[c * 128:(c + 1) * 128, :]
        cumb = jnp.dot(ltri, blk, preferred_element_type=jnp.float32) + offs
        parts.append(cumb)
        offs = cumb[127:128, :]
    cum = jnp.concatenate(parts, axis=0)

    counts = cum[s - 1:s, :]
    pc = jnp.ceil(counts / _TILE) * _TILE
    e1i = lax.broadcasted_iota(jnp.int32, (e, e), 0)
    e2i = lax.broadcasted_iota(jnp.int32, (e, e), 1)
    slt = jnp.where(e1i < e2i, 1.0, 0.0)
    aoff = jnp.dot(pc, slt, preferred_element_type=jnp.float32)

    g1 = jnp.sum(jnp.where(oh1, aoff + cum, 0.0), axis=-1, keepdims=True) - 1.0
    g2 = jnp.sum(jnp.where(oh2, aoff + cum, 0.0), axis=-1, keepdims=True) - 1.0
    pos0_ref[...] = g1.astype(jnp.int32)
    pos1_ref[...] = g2.astype(jnp.int32)

    endt = (aoff + pc) / _TILE
    eye = e1i == e2i
    end_col = jnp.sum(jnp.where(eye, jnp.broadcast_to(endt, (e, e)), 0.0),
                      axis=-1, keepdims=True)
    ti = lax.broadcasted_iota(jnp.int32, (e, 64), 1).astype(jnp.float32) - 1.0
    full_before = jnp.sum(jnp.where(ti >= end_col, 1.0, 0.0), axis=0,
                          keepdims=True)
    etile = jnp.minimum(full_before, float(e - 1))
    ntile = jnp.sum(pc, axis=-1, keepdims=True) / _TILE
    c64 = lax.broadcasted_iota(jnp.int32, (1, 64), 1)
    meta_ref[...] = jnp.where(c64 == 0, ntile, etile).astype(jnp.int32)


@functools.cache
def _get_dispatch():
    mesh = plsc.VectorSubcoreMesh(core_axis_name="c", subcore_axis_name="s")

    @functools.partial(
        pl.kernel,
        mesh=mesh,
        out_type=jax.ShapeDtypeStruct((_CAP, _D), jnp.float32),
        scratch_types=[
            pltpu.VMEM((_TPW,), jnp.int32),
            pltpu.VMEM((_TPW,), jnp.int32),
            pltpu.VMEM((_TPW, _D), jnp.float32),
            pltpu.SemaphoreType.DMA,
            pltpu.SemaphoreType.DMA,
        ],
    )
    def _dispatch(x_hbm, pos0_hbm, pos1_hbm, xs_hbm, idx0_v, idx1_v, rows_v,
                  sem0, sem1):
        wid = lax.axis_index("s") * 2 + lax.axis_index("c")
        base = wid * _TPW
        pltpu.sync_copy(x_hbm.at[pl.ds(base, _TPW)], rows_v)
        pltpu.sync_copy(pos0_hbm.at[pl.ds(base, _TPW)], idx0_v)
        pltpu.sync_copy(pos1_hbm.at[pl.ds(base, _TPW)], idx1_v)
        c0 = pltpu.async_copy(rows_v, xs_hbm.at[idx0_v], sem0)
        c1 = pltpu.async_copy(rows_v, xs_hbm.at[idx1_v], sem1)
        c0.wait()
        c1.wait()

    return _dispatch


def _expert_kernel(meta_ref, xs_ref, w1a_ref, w1b_ref, b1a_ref, b1b_ref,
                   w2_ref, b2_ref, ys_ref):
    j = pl.program_id(0)
    i = pl.program_id(1)

    @pl.when(i < meta_ref[0])
    def _compute():
        xt = xs_ref[pl.ds(i * _TILE, _TILE), :]
        h1 = jnp.dot(xt, w1a_ref[0], preferred_element_type=jnp.float32)
        h1 = h1 + b1a_ref[0]
        h2 = jnp.dot(xt, w1b_ref[0], preferred_element_type=jnp.float32)
        h2 = h2 + b1b_ref[0]
        g = h1 * _gelu(h2)
        partial = jnp.dot(g, w2_ref[0], preferred_element_type=jnp.float32)

        @pl.when(j == 0)
        def _set():
            ys_ref[pl.ds(i * _TILE, _TILE), :] = partial

        @pl.when(j > 0)
        def _acc():
            ys_ref[pl.ds(i * _TILE, _TILE), :] += partial

        @pl.when(j == _NF - 1)
        def _bias():
            ys_ref[pl.ds(i * _TILE, _TILE), :] += b2_ref[0]


@functools.cache
def _get_combine():
    mesh = plsc.VectorSubcoreMesh(core_axis_name="c", subcore_axis_name="s")

    @functools.partial(
        pl.kernel,
        mesh=mesh,
        out_type=jax.ShapeDtypeStruct((_S, _D), jnp.float32),
        scratch_types=[
            pltpu.VMEM((_TPW,), jnp.int32),
            pltpu.VMEM((_TPW,), jnp.int32),
            pltpu.VMEM((_TPW, _LANES), jnp.float32),
            pltpu.VMEM((_TPW, _LANES), jnp.float32),
            pltpu.VMEM((_TPW, _D), jnp.float32),
            pltpu.VMEM((_TPW, _D), jnp.float32),
            pltpu.SemaphoreType.DMA,
            pltpu.SemaphoreType.DMA,
        ],
    )
    def _combine(ys_hbm, pos0_hbm, pos1_hbm, w0_hbm, w1_hbm, out_hbm,
                 idx0_v, idx1_v, w0_v, w1_v, rows0_v, rows1_v, sem0, sem1):
        wid = lax.axis_index("s") * 2 + lax.axis_index("c")
        base = wid * _TPW
        pltpu.sync_copy(pos0_hbm.at[pl.ds(base, _TPW)], idx0_v)
        pltpu.sync_copy(pos1_hbm.at[pl.ds(base, _TPW)], idx1_v)
        c0 = pltpu.async_copy(ys_hbm.at[idx0_v], rows0_v, sem0)
        c1 = pltpu.async_copy(ys_hbm.at[idx1_v], rows1_v, sem1)
        pltpu.sync_copy(w0_hbm.at[pl.ds(base, _TPW)], w0_v)
        pltpu.sync_copy(w1_hbm.at[pl.ds(base, _TPW)], w1_v)
        c0.wait()
        c1.wait()

        def body(t, carry):
            wv0 = w0_v[t, pl.ds(0, _LANES)]
            wv1 = w1_v[t, pl.ds(0, _LANES)]
            for cc in range(_D // _LANES):
                a = rows0_v[t, pl.ds(cc * _LANES, _LANES)]
                b = rows1_v[t, pl.ds(cc * _LANES, _LANES)]
                rows0_v[t, pl.ds(cc * _LANES, _LANES)] = wv0 * a + wv1 * b
            return carry

        lax.fori_loop(0, _TPW, body, 0)
        pltpu.sync_copy(rows0_v, out_hbm.at[pl.ds(base, _TPW)])

    return _combine


@jax.jit
def kernel(x, r_w1, r_b1, ln_scale, ln_bias, r_w2, r_b2, ew1, eb1, ew2, eb2,
           expert_priors):
    del expert_priors
    b, s, d = x.shape
    x2d = x.reshape(s, d)

    pos0, pos1, w0, w1, meta = pl.pallas_call(
        _router_kernel,
        out_shape=[
            jax.ShapeDtypeStruct((s, 1), jnp.int32),
            jax.ShapeDtypeStruct((s, 1), jnp.int32),
            jax.ShapeDtypeStruct((s, _LANES), jnp.float32),
            jax.ShapeDtypeStruct((s, _LANES), jnp.float32),
            jax.ShapeDtypeStruct((1, 64), jnp.int32),
        ],
    )(x2d, r_w1, r_b1.reshape(1, -1), ln_scale.reshape(1, -1),
      ln_bias.reshape(1, -1), r_w2, r_b2.reshape(1, -1))

    pos0f = pos0.reshape(s)
    pos1f = pos1.reshape(s)
    meta64 = meta.reshape(64)

    xs = _get_dispatch()(x2d, pos0f, pos1f)

    eb1_3d = eb1.reshape(_E, 1, 2 * _F)
    eb2_3d = eb2.reshape(_E, 1, _D)
    grid_spec = pltpu.PrefetchScalarGridSpec(
        num_scalar_prefetch=1,
        grid=(_NF, _TMAX),
        in_specs=[
            pl.BlockSpec((_CAP, _D), lambda j, i, m: (0, 0)),
            pl.BlockSpec((1, _D, _FBLK), lambda j, i, m: (m[1 + i], 0, j)),
            pl.BlockSpec((1, _D, _FBLK), lambda j, i, m: (m[1 + i], 0, j + _NF)),
            pl.BlockSpec((1, 1, _FBLK), lambda j, i, m: (m[1 + i], 0, j)),
            pl.BlockSpec((1, 1, _FBLK), lambda j, i, m: (m[1 + i], 0, j + _NF)),
            pl.BlockSpec((1, _FBLK, _D), lambda j, i, m: (m[1 + i], j, 0)),
            pl.BlockSpec((1, 1, _D), lambda j, i, m: (m[1 + i], 0, 0)),
        ],
        out_specs=pl.BlockSpec((_CAP, _D), lambda j, i, m: (0, 0)),
    )
    ys = pl.pallas_call(
        _expert_kernel,
        grid_spec=grid_spec,
        out_shape=jax.ShapeDtypeStruct((_CAP, _D), jnp.float32),
        compiler_params=pltpu.CompilerParams(
            dimension_semantics=("arbitrary", "arbitrary"),
        ),
    )(meta64, xs, ew1, ew1, eb1_3d, eb1_3d, ew2, eb2_3d)

    out = _get_combine()(ys, pos0f, pos1f, w0, w1)

    return (out.reshape(b, s, d), 0.0)


# speedup vs baseline: 1.6193x; 1.1903x over previous
import functools

import jax
import jax.numpy as jnp
from jax import lax
from jax.experimental import pallas as pl
from jax.experimental.pallas import tpu as pltpu
from jax.experimental.pallas import tpu_sc as plsc

_D = 768
_F = 3072
_E = 8
_S = 2048
_TILE = 128
_TMAX = 40
_CAP = _TMAX * _TILE
_FBLK = 1024
_NF = _F // _FBLK
_NW = 32
_TPW = _S // _NW
_LANES = 16


def _gelu(x):
    sqrt_2_pi = 0.7978845608028654
    coef = 0.044715
    x3 = x ** 3
    inner = sqrt_2_pi * (x + coef * x3)
    return 0.5 * x * (1.0 + jnp.tanh(inner))


def _router_kernel(x_ref, w1_ref, b1_ref, lns_ref, lnb_ref, w2_ref, b2_ref,
                   pos0_ref, pos1_ref, w0_ref, w1o_ref, meta_ref):
    x = x_ref[...]
    h = jnp.dot(x, w1_ref[...], preferred_element_type=jnp.float32)
    h = h + b1_ref[...]
    mean = jnp.mean(h, axis=-1, keepdims=True)
    var = jnp.mean(jnp.square(h - mean), axis=-1, keepdims=True)
    h = (h - mean) * lax.rsqrt(var + 1e-6) * lns_ref[...] + lnb_ref[...]
    h = _gelu(h)
    logits = jnp.dot(h, w2_ref[...], preferred_element_type=jnp.float32)
    logits = logits + b2_ref[...]
    lmax = jnp.max(logits, axis=-1, keepdims=True)
    ex = jnp.exp(logits - lmax)
    p = ex / jnp.sum(ex, axis=-1, keepdims=True)

    s, e = p.shape
    ei = lax.broadcasted_iota(jnp.int32, (s, e), 1)
    m1 = jnp.max(p, axis=-1, keepdims=True)
    i1 = jnp.min(jnp.where(p == m1, ei, e), axis=-1, keepdims=True)
    oh1 = ei == i1
    pm = jnp.where(oh1, -jnp.inf, p)
    m2 = jnp.max(pm, axis=-1, keepdims=True)
    i2 = jnp.min(jnp.where(pm == m2, ei, e), axis=-1, keepdims=True)
    oh2 = ei == i2
    denom = m1 + m2
    w0_ref[...] = jnp.broadcast_to(m1 / denom, (s, _LANES))
    w1o_ref[...] = jnp.broadcast_to(m2 / denom, (s, _LANES))

    m01 = jnp.where(oh1 | oh2, 1.0, 0.0)
    l1 = lax.broadcasted_iota(jnp.int32, (128, 128), 0)
    l2 = lax.broadcasted_iota(jnp.int32, (128, 128), 1)
    ltri = jnp.where(l2 <= l1, 1.0, 0.0)
    offs = jnp.zeros((1, e), jnp.float32)
    parts = []
    for c in range(s // 128):
        blk = m01[c * 128:(c + 1) * 128, :]
        cumb = jnp.dot(ltri, blk, preferred_element_type=jnp.float32) + offs
        parts.append(cumb)
        offs = cumb[127:128, :]
    cum = jnp.concatenate(parts, axis=0)

    counts = cum[s - 1:s, :]
    pc = jnp.ceil(counts / _TILE) * _TILE
    e1i = lax.broadcasted_iota(jnp.int32, (e, e), 0)
    e2i = lax.broadcasted_iota(jnp.int32, (e, e), 1)
    slt = jnp.where(e1i < e2i, 1.0, 0.0)
    aoff = jnp.dot(pc, slt, preferred_element_type=jnp.float32)

    g1 = jnp.sum(jnp.where(oh1, aoff + cum, 0.0), axis=-1, keepdims=True) - 1.0
    g2 = jnp.sum(jnp.where(oh2, aoff + cum, 0.0), axis=-1, keepdims=True) - 1.0
    pos0_ref[...] = g1.astype(jnp.int32)
    pos1_ref[...] = g2.astype(jnp.int32)

    endt = (aoff + pc) / _TILE
    eye = e1i == e2i
    end_col = jnp.sum(jnp.where(eye, jnp.broadcast_to(endt, (e, e)), 0.0),
                      axis=-1, keepdims=True)
    ti = lax.broadcasted_iota(jnp.int32, (e, 64), 1).astype(jnp.float32) - 1.0
    full_before = jnp.sum(jnp.where(ti >= end_col, 1.0, 0.0), axis=0,
                          keepdims=True)
    etile = jnp.minimum(full_before, float(e - 1))
    ntile = jnp.sum(pc, axis=-1, keepdims=True) / _TILE
    c64 = lax.broadcasted_iota(jnp.int32, (1, 64), 1)
    meta_ref[...] = jnp.where(c64 == 0, ntile, etile).astype(jnp.int32)


@functools.cache
def _get_dispatch():
    mesh = plsc.VectorSubcoreMesh(core_axis_name="c", subcore_axis_name="s")

    @functools.partial(
        pl.kernel,
        mesh=mesh,
        out_type=jax.ShapeDtypeStruct((_CAP, _D), jnp.float32),
        scratch_types=[
            pltpu.VMEM((_TPW,), jnp.int32),
            pltpu.VMEM((_TPW,), jnp.int32),
            pltpu.VMEM((_TPW, _D), jnp.float32),
            pltpu.SemaphoreType.DMA,
            pltpu.SemaphoreType.DMA,
        ],
    )
    def _dispatch(x_hbm, pos0_hbm, pos1_hbm, xs_hbm, idx0_v, idx1_v, rows_v,
                  sem0, sem1):
        wid = lax.axis_index("s") * 2 + lax.axis_index("c")
        base = wid * _TPW
        pltpu.sync_copy(x_hbm.at[pl.ds(base, _TPW)], rows_v)
        pltpu.sync_copy(pos0_hbm.at[pl.ds(base, _TPW)], idx0_v)
        pltpu.sync_copy(pos1_hbm.at[pl.ds(base, _TPW)], idx1_v)
        c0 = pltpu.async_copy(rows_v, xs_hbm.at[idx0_v], sem0)
        c1 = pltpu.async_copy(rows_v, xs_hbm.at[idx1_v], sem1)
        c0.wait()
        c1.wait()

    return _dispatch


def _expert_kernel(meta_ref, xs_ref, w1a_ref, w1b_ref, b1a_ref, b1b_ref,
                   w2_ref, b2_ref, ys_ref):
    i = pl.program_id(0)

    @pl.when(i < meta_ref[0])
    def _compute():
        xt = xs_ref[...]
        h1 = jnp.dot(xt, w1a_ref[0], preferred_element_type=jnp.float32)
        h1 = h1 + b1a_ref[0]
        h2 = jnp.dot(xt, w1b_ref[0], preferred_element_type=jnp.float32)
        h2 = h2 + b1b_ref[0]
        g = h1 * _gelu(h2)
        out = jnp.dot(g, w2_ref[0], preferred_element_type=jnp.float32)
        ys_ref[...] = out + b2_ref[0]


@functools.cache
def _get_combine():
    mesh = plsc.VectorSubcoreMesh(core_axis_name="c", subcore_axis_name="s")

    @functools.partial(
        pl.kernel,
        mesh=mesh,
        out_type=jax.ShapeDtypeStruct((_S, _D), jnp.float32),
        scratch_types=[
            pltpu.VMEM((_TPW,), jnp.int32),
            pltpu.VMEM((_TPW,), jnp.int32),
            pltpu.VMEM((_TPW, _LANES), jnp.float32),
            pltpu.VMEM((_TPW, _LANES), jnp.float32),
            pltpu.VMEM((_TPW, _D), jnp.float32),
            pltpu.VMEM((_TPW, _D), jnp.float32),
            pltpu.SemaphoreType.DMA,
            pltpu.SemaphoreType.DMA,
        ],
    )
    def _combine(ys_hbm, pos0_hbm, pos1_hbm, w0_hbm, w1_hbm, out_hbm,
                 idx0_v, idx1_v, w0_v, w1_v, rows0_v, rows1_v, sem0, sem1):
        wid = lax.axis_index("s") * 2 + lax.axis_index("c")
        base = wid * _TPW
        pltpu.sync_copy(pos0_hbm.at[pl.ds(base, _TPW)], idx0_v)
        pltpu.sync_copy(pos1_hbm.at[pl.ds(base, _TPW)], idx1_v)
        c0 = pltpu.async_copy(ys_hbm.at[idx0_v], rows0_v, sem0)
        c1 = pltpu.async_copy(ys_hbm.at[idx1_v], rows1_v, sem1)
        pltpu.sync_copy(w0_hbm.at[pl.ds(base, _TPW)], w0_v)
        pltpu.sync_copy(w1_hbm.at[pl.ds(base, _TPW)], w1_v)
        c0.wait()
        c1.wait()

        def body(t, carry):
            wv0 = w0_v[t, pl.ds(0, _LANES)]
            wv1 = w1_v[t, pl.ds(0, _LANES)]
            for cc in range(_D // _LANES):
                a = rows0_v[t, pl.ds(cc * _LANES, _LANES)]
                b = rows1_v[t, pl.ds(cc * _LANES, _LANES)]
                rows0_v[t, pl.ds(cc * _LANES, _LANES)] = wv0 * a + wv1 * b
            return carry

        lax.fori_loop(0, _TPW, body, 0)
        pltpu.sync_copy(rows0_v, out_hbm.at[pl.ds(base, _TPW)])

    return _combine


@jax.jit
def kernel(x, r_w1, r_b1, ln_scale, ln_bias, r_w2, r_b2, ew1, eb1, ew2, eb2,
           expert_priors):
    del expert_priors
    b, s, d = x.shape
    x2d = x.reshape(s, d)

    pos0, pos1, w0, w1, meta = pl.pallas_call(
        _router_kernel,
        out_shape=[
            jax.ShapeDtypeStruct((s, 1), jnp.int32),
            jax.ShapeDtypeStruct((s, 1), jnp.int32),
            jax.ShapeDtypeStruct((s, _LANES), jnp.float32),
            jax.ShapeDtypeStruct((s, _LANES), jnp.float32),
            jax.ShapeDtypeStruct((1, 64), jnp.int32),
        ],
    )(x2d, r_w1, r_b1.reshape(1, -1), ln_scale.reshape(1, -1),
      ln_bias.reshape(1, -1), r_w2, r_b2.reshape(1, -1))

    pos0f = pos0.reshape(s)
    pos1f = pos1.reshape(s)
    meta64 = meta.reshape(64)

    xs = _get_dispatch()(x2d, pos0f, pos1f)

    eb1_3d = eb1.reshape(_E, 1, 2 * _F)
    eb2_3d = eb2.reshape(_E, 1, _D)
    grid_spec = pltpu.PrefetchScalarGridSpec(
        num_scalar_prefetch=1,
        grid=(_TMAX,),
        in_specs=[
            pl.BlockSpec((_TILE, _D), lambda i, m: (i, 0)),
            pl.BlockSpec((1, _D, _F), lambda i, m: (m[1 + i], 0, 0)),
            pl.BlockSpec((1, _D, _F), lambda i, m: (m[1 + i], 0, 1)),
            pl.BlockSpec((1, 1, _F), lambda i, m: (m[1 + i], 0, 0)),
            pl.BlockSpec((1, 1, _F), lambda i, m: (m[1 + i], 0, 1)),
            pl.BlockSpec((1, _F, _D), lambda i, m: (m[1 + i], 0, 0)),
            pl.BlockSpec((1, 1, _D), lambda i, m: (m[1 + i], 0, 0)),
        ],
        out_specs=pl.BlockSpec((_TILE, _D), lambda i, m: (i, 0)),
    )
    ys = pl.pallas_call(
        _expert_kernel,
        grid_spec=grid_spec,
        out_shape=jax.ShapeDtypeStruct((_CAP, _D), jnp.float32),
        compiler_params=pltpu.CompilerParams(
            dimension_semantics=("arbitrary",),
            vmem_limit_bytes=100 * 1024 * 1024,
        ),
    )(meta64, xs, ew1, ew1, eb1_3d, eb1_3d, ew2, eb2_3d)

    out = _get_combine()(ys, pos0f, pos1f, w0, w1)

    return (out.reshape(b, s, d), 0.0)


# speedup vs baseline: 1.8244x; 1.1266x over previous
import functools

import jax
import jax.numpy as jnp
from jax import lax
from jax.experimental import pallas as pl
from jax.experimental.pallas import tpu as pltpu
from jax.experimental.pallas import tpu_sc as plsc

_D = 768
_F = 3072
_E = 8
_S = 2048
_TILE = 256
_TMAX = 24
_CAP = _TMAX * _TILE
_FBLK = 1024
_NF = _F // _FBLK
_NW = 32
_TPW = _S // _NW
_LANES = 16


def _gelu(x):
    sqrt_2_pi = 0.7978845608028654
    coef = 0.044715
    x3 = x ** 3
    inner = sqrt_2_pi * (x + coef * x3)
    return 0.5 * x * (1.0 + jnp.tanh(inner))


def _router_kernel(x_ref, w1_ref, b1_ref, lns_ref, lnb_ref, w2_ref, b2_ref,
                   pos0_ref, pos1_ref, w0_ref, w1o_ref, meta_ref):
    x = x_ref[...]
    h = jnp.dot(x, w1_ref[...], preferred_element_type=jnp.float32)
    h = h + b1_ref[...]
    mean = jnp.mean(h, axis=-1, keepdims=True)
    var = jnp.mean(jnp.square(h - mean), axis=-1, keepdims=True)
    h = (h - mean) * lax.rsqrt(var + 1e-6) * lns_ref[...] + lnb_ref[...]
    h = _gelu(h)
    logits = jnp.dot(h, w2_ref[...], preferred_element_type=jnp.float32)
    logits = logits + b2_ref[...]
    lmax = jnp.max(logits, axis=-1, keepdims=True)
    ex = jnp.exp(logits - lmax)
    p = ex / jnp.sum(ex, axis=-1, keepdims=True)

    s, e = p.shape
    ei = lax.broadcasted_iota(jnp.int32, (s, e), 1)
    m1 = jnp.max(p, axis=-1, keepdims=True)
    i1 = jnp.min(jnp.where(p == m1, ei, e), axis=-1, keepdims=True)
    oh1 = ei == i1
    pm = jnp.where(oh1, -jnp.inf, p)
    m2 = jnp.max(pm, axis=-1, keepdims=True)
    i2 = jnp.min(jnp.where(pm == m2, ei, e), axis=-1, keepdims=True)
    oh2 = ei == i2
    denom = m1 + m2
    w0_ref[...] = jnp.broadcast_to(m1 / denom, (s, _LANES))
    w1o_ref[...] = jnp.broadcast_to(m2 / denom, (s, _LANES))

    m01 = jnp.where(oh1 | oh2, 1.0, 0.0)
    l1 = lax.broadcasted_iota(jnp.int32, (128, 128), 0)
    l2 = lax.broadcasted_iota(jnp.int32, (128, 128), 1)
    ltri = jnp.where(l2 <= l1, 1.0, 0.0)
    offs = jnp.zeros((1, e), jnp.float32)
    parts = []
    for c in range(s // 128):
        blk = m01[c * 128:(c + 1) * 128, :]
        cumb = jnp.dot(ltri, blk, preferred_element_type=jnp.float32) + offs
        parts.append(cumb)
        offs = cumb[127:128, :]
    cum = jnp.concatenate(parts, axis=0)

    counts = cum[s - 1:s, :]
    pc = jnp.ceil(counts / _TILE) * _TILE
    e1i = lax.broadcasted_iota(jnp.int32, (e, e), 0)
    e2i = lax.broadcasted_iota(jnp.int32, (e, e), 1)
    slt = jnp.where(e1i < e2i, 1.0, 0.0)
    aoff = jnp.dot(pc, slt, preferred_element_type=jnp.float32)

    g1 = jnp.sum(jnp.where(oh1, aoff + cum, 0.0), axis=-1, keepdims=True) - 1.0
    g2 = jnp.sum(jnp.where(oh2, aoff + cum, 0.0), axis=-1, keepdims=True) - 1.0
    pos0_ref[...] = g1.astype(jnp.int32)
    pos1_ref[...] = g2.astype(jnp.int32)

    endt = (aoff + pc) / _TILE
    eye = e1i == e2i
    end_col = jnp.sum(jnp.where(eye, jnp.broadcast_to(endt, (e, e)), 0.0),
                      axis=-1, keepdims=True)
    ti = lax.broadcasted_iota(jnp.int32, (e, 64), 1).astype(jnp.float32) - 1.0
    full_before = jnp.sum(jnp.where(ti >= end_col, 1.0, 0.0), axis=0,
                          keepdims=True)
    etile = jnp.minimum(full_before, float(e - 1))
    ntile = jnp.sum(pc, axis=-1, keepdims=True) / _TILE
    c64 = lax.broadcasted_iota(jnp.int32, (1, 64), 1)
    meta_ref[...] = jnp.where(c64 == 0, ntile, etile).astype(jnp.int32)


@functools.cache
def _get_dispatch():
    mesh = plsc.VectorSubcoreMesh(core_axis_name="c", subcore_axis_name="s")

    @functools.partial(
        pl.kernel,
        mesh=mesh,
        out_type=jax.ShapeDtypeStruct((_CAP, _D), jnp.float32),
        scratch_types=[
            pltpu.VMEM((_TPW,), jnp.int32),
            pltpu.VMEM((_TPW,), jnp.int32),
            pltpu.VMEM((_TPW, _D), jnp.float32),
            pltpu.SemaphoreType.DMA,
            pltpu.SemaphoreType.DMA,
        ],
    )
    def _dispatch(x_hbm, pos0_hbm, pos1_hbm, xs_hbm, idx0_v, idx1_v, rows_v,
                  sem0, sem1):
        wid = lax.axis_index("s") * 2 + lax.axis_index("c")
        base = wid * _TPW
        pltpu.sync_copy(x_hbm.at[pl.ds(base, _TPW)], rows_v)
        pltpu.sync_copy(pos0_hbm.at[pl.ds(base, _TPW)], idx0_v)
        pltpu.sync_copy(pos1_hbm.at[pl.ds(base, _TPW)], idx1_v)
        c0 = pltpu.async_copy(rows_v, xs_hbm.at[idx0_v], sem0)
        c1 = pltpu.async_copy(rows_v, xs_hbm.at[idx1_v], sem1)
        c0.wait()
        c1.wait()

    return _dispatch


def _expert_kernel(meta_ref, xs_ref, w1a_ref, w1b_ref, b1a_ref, b1b_ref,
                   w2_ref, b2_ref, ys_ref):
    i = pl.program_id(0)

    @pl.when(i < meta_ref[0])
    def _compute():
        xt = xs_ref[...]
        h1 = jnp.dot(xt, w1a_ref[0], preferred_element_type=jnp.float32)
        h1 = h1 + b1a_ref[0]
        h2 = jnp.dot(xt, w1b_ref[0], preferred_element_type=jnp.float32)
        h2 = h2 + b1b_ref[0]
        g = h1 * _gelu(h2)
        out = jnp.dot(g, w2_ref[0], preferred_element_type=jnp.float32)
        ys_ref[...] = out + b2_ref[0]


@functools.cache
def _get_combine():
    mesh = plsc.VectorSubcoreMesh(core_axis_name="c", subcore_axis_name="s")

    @functools.partial(
        pl.kernel,
        mesh=mesh,
        out_type=jax.ShapeDtypeStruct((_S, _D), jnp.float32),
        scratch_types=[
            pltpu.VMEM((_TPW,), jnp.int32),
            pltpu.VMEM((_TPW,), jnp.int32),
            pltpu.VMEM((_TPW, _LANES), jnp.float32),
            pltpu.VMEM((_TPW, _LANES), jnp.float32),
            pltpu.VMEM((_TPW, _D), jnp.float32),
            pltpu.VMEM((_TPW, _D), jnp.float32),
            pltpu.SemaphoreType.DMA,
            pltpu.SemaphoreType.DMA,
        ],
    )
    def _combine(ys_hbm, pos0_hbm, pos1_hbm, w0_hbm, w1_hbm, out_hbm,
                 idx0_v, idx1_v, w0_v, w1_v, rows0_v, rows1_v, sem0, sem1):
        wid = lax.axis_index("s") * 2 + lax.axis_index("c")
        base = wid * _TPW
        pltpu.sync_copy(pos0_hbm.at[pl.ds(base, _TPW)], idx0_v)
        pltpu.sync_copy(pos1_hbm.at[pl.ds(base, _TPW)], idx1_v)
        c0 = pltpu.async_copy(ys_hbm.at[idx0_v], rows0_v, sem0)
        c1 = pltpu.async_copy(ys_hbm.at[idx1_v], rows1_v, sem1)
        pltpu.sync_copy(w0_hbm.at[pl.ds(base, _TPW)], w0_v)
        pltpu.sync_copy(w1_hbm.at[pl.ds(base, _TPW)], w1_v)
        c0.wait()
        c1.wait()

        def body(t, carry):
            wv0 = w0_v[t, pl.ds(0, _LANES)]
            wv1 = w1_v[t, pl.ds(0, _LANES)]
            for cc in range(_D // _LANES):
                a = rows0_v[t, pl.ds(cc * _LANES, _LANES)]
                b = rows1_v[t, pl.ds(cc * _LANES, _LANES)]
                rows0_v[t, pl.ds(cc * _LANES, _LANES)] = wv0 * a + wv1 * b
            return carry

        lax.fori_loop(0, _TPW, body, 0)
        pltpu.sync_copy(rows0_v, out_hbm.at[pl.ds(base, _TPW)])

    return _combine


@jax.jit
def kernel(x, r_w1, r_b1, ln_scale, ln_bias, r_w2, r_b2, ew1, eb1, ew2, eb2,
           expert_priors):
    del expert_priors
    b, s, d = x.shape
    x2d = x.reshape(s, d)

    pos0, pos1, w0, w1, meta = pl.pallas_call(
        _router_kernel,
        out_shape=[
            jax.ShapeDtypeStruct((s, 1), jnp.int32),
            jax.ShapeDtypeStruct((s, 1), jnp.int32),
            jax.ShapeDtypeStruct((s, _LANES), jnp.float32),
            jax.ShapeDtypeStruct((s, _LANES), jnp.float32),
            jax.ShapeDtypeStruct((1, 64), jnp.int32),
        ],
    )(x2d, r_w1, r_b1.reshape(1, -1), ln_scale.reshape(1, -1),
      ln_bias.reshape(1, -1), r_w2, r_b2.reshape(1, -1))

    pos0f = pos0.reshape(s)
    pos1f = pos1.reshape(s)
    meta64 = meta.reshape(64)

    xs = _get_dispatch()(x2d, pos0f, pos1f)

    eb1_3d = eb1.reshape(_E, 1, 2 * _F)
    eb2_3d = eb2.reshape(_E, 1, _D)
    grid_spec = pltpu.PrefetchScalarGridSpec(
        num_scalar_prefetch=1,
        grid=(_TMAX,),
        in_specs=[
            pl.BlockSpec((_TILE, _D), lambda i, m: (i, 0)),
            pl.BlockSpec((1, _D, _F), lambda i, m: (m[1 + i], 0, 0)),
            pl.BlockSpec((1, _D, _F), lambda i, m: (m[1 + i], 0, 1)),
            pl.BlockSpec((1, 1, _F), lambda i, m: (m[1 + i], 0, 0)),
            pl.BlockSpec((1, 1, _F), lambda i, m: (m[1 + i], 0, 1)),
            pl.BlockSpec((1, _F, _D), lambda i, m: (m[1 + i], 0, 0)),
            pl.BlockSpec((1, 1, _D), lambda i, m: (m[1 + i], 0, 0)),
        ],
        out_specs=pl.BlockSpec((_TILE, _D), lambda i, m: (i, 0)),
    )
    ys = pl.pallas_call(
        _expert_kernel,
        grid_spec=grid_spec,
        out_shape=jax.ShapeDtypeStruct((_CAP, _D), jnp.float32),
        compiler_params=pltpu.CompilerParams(
            dimension_semantics=("arbitrary",),
            vmem_limit_bytes=100 * 1024 * 1024,
        ),
    )(meta64, xs, ew1, ew1, eb1_3d, eb1_3d, ew2, eb2_3d)

    out = _get_combine()(ys, pos0f, pos1f, w0, w1)

    return (out.reshape(b, s, d), 0.0)


# speedup vs baseline: 1.8296x; 1.0029x over previous
import functools

import jax
import jax.numpy as jnp
from jax import lax
from jax.experimental import pallas as pl
from jax.experimental.pallas import tpu as pltpu
from jax.experimental.pallas import tpu_sc as plsc

_D = 768
_F = 3072
_E = 8
_S = 2048
_TILE = 256
_TMAX = 24
_CAP = _TMAX * _TILE
_FBLK = 1024
_NF = _F // _FBLK
_NW = 32
_TPW = _S // _NW
_LANES = 16


def _gelu(x):
    sqrt_2_pi = 0.7978845608028654
    coef = 0.044715
    x3 = x ** 3
    inner = sqrt_2_pi * (x + coef * x3)
    return 0.5 * x * (1.0 + jnp.tanh(inner))


def _router_kernel(x_ref, w1_ref, b1_ref, lns_ref, lnb_ref, w2_ref, b2_ref,
                   pos0_ref, pos1_ref, w0_ref, w1o_ref, meta_ref):
    x = x_ref[...]
    h = jnp.dot(x, w1_ref[...], preferred_element_type=jnp.float32)
    h = h + b1_ref[...]
    mean = jnp.mean(h, axis=-1, keepdims=True)
    var = jnp.mean(jnp.square(h - mean), axis=-1, keepdims=True)
    h = (h - mean) * lax.rsqrt(var + 1e-6) * lns_ref[...] + lnb_ref[...]
    h = _gelu(h)
    logits = jnp.dot(h, w2_ref[...], preferred_element_type=jnp.float32)
    logits = logits + b2_ref[...]
    lmax = jnp.max(logits, axis=-1, keepdims=True)
    ex = jnp.exp(logits - lmax)
    p = ex / jnp.sum(ex, axis=-1, keepdims=True)

    s, e = p.shape
    ei = lax.broadcasted_iota(jnp.int32, (s, e), 1)
    m1 = jnp.max(p, axis=-1, keepdims=True)
    i1 = jnp.min(jnp.where(p == m1, ei, e), axis=-1, keepdims=True)
    oh1 = ei == i1
    pm = jnp.where(oh1, -jnp.inf, p)
    m2 = jnp.max(pm, axis=-1, keepdims=True)
    i2 = jnp.min(jnp.where(pm == m2, ei, e), axis=-1, keepdims=True)
    oh2 = ei == i2
    denom = m1 + m2
    w0_ref[...] = jnp.broadcast_to(m1 / denom, (s, _LANES))
    w1o_ref[...] = jnp.broadcast_to(m2 / denom, (s, _LANES))

    m01 = jnp.where(oh1 | oh2, 1.0, 0.0)
    l1 = lax.broadcasted_iota(jnp.int32, (128, 128), 0)
    l2 = lax.broadcasted_iota(jnp.int32, (128, 128), 1)
    ltri = jnp.where(l2 <= l1, 1.0, 0.0)
    nchunk = s // 128
    parts = []
    lasts = []
    for c in range(nchunk):
        blk = m01[c * 128:(c + 1) * 128, :]
        cumb = jnp.dot(ltri, blk, preferred_element_type=jnp.float32)
        parts.append(cumb)
        lasts.append(cumb[127:128, :])
    last = jnp.concatenate(lasts, axis=0)
    k1 = lax.broadcasted_iota(jnp.int32, (nchunk, nchunk), 0)
    k2 = lax.broadcasted_iota(jnp.int32, (nchunk, nchunk), 1)
    lo = jnp.where(k2 < k1, 1.0, 0.0)
    offs = jnp.dot(lo, last, preferred_element_type=jnp.float32)
    cum = jnp.concatenate(
        [parts[c] + offs[c:c + 1, :] for c in range(nchunk)], axis=0)

    counts = cum[s - 1:s, :]
    pc = jnp.ceil(counts / _TILE) * _TILE
    e1i = lax.broadcasted_iota(jnp.int32, (e, e), 0)
    e2i = lax.broadcasted_iota(jnp.int32, (e, e), 1)
    slt = jnp.where(e1i < e2i, 1.0, 0.0)
    aoff = jnp.dot(pc, slt, preferred_element_type=jnp.float32)

    g1 = jnp.sum(jnp.where(oh1, aoff + cum, 0.0), axis=-1, keepdims=True) - 1.0
    g2 = jnp.sum(jnp.where(oh2, aoff + cum, 0.0), axis=-1, keepdims=True) - 1.0
    pos0_ref[...] = g1.astype(jnp.int32)
    pos1_ref[...] = g2.astype(jnp.int32)

    endt = (aoff + pc) / _TILE
    eye = e1i == e2i
    end_col = jnp.sum(jnp.where(eye, jnp.broadcast_to(endt, (e, e)), 0.0),
                      axis=-1, keepdims=True)
    ti = lax.broadcasted_iota(jnp.int32, (e, 64), 1).astype(jnp.float32) - 1.0
    full_before = jnp.sum(jnp.where(ti >= end_col, 1.0, 0.0), axis=0,
                          keepdims=True)
    etile = jnp.minimum(full_before, float(e - 1))
    ntile = jnp.sum(pc, axis=-1, keepdims=True) / _TILE
    c64 = lax.broadcasted_iota(jnp.int32, (1, 64), 1)
    meta_ref[...] = jnp.where(c64 == 0, ntile, etile).astype(jnp.int32)


@functools.cache
def _get_dispatch():
    mesh = plsc.VectorSubcoreMesh(core_axis_name="c", subcore_axis_name="s")

    @functools.partial(
        pl.kernel,
        mesh=mesh,
        out_type=jax.ShapeDtypeStruct((_CAP, _D), jnp.float32),
        scratch_types=[
            pltpu.VMEM((_TPW,), jnp.int32),
            pltpu.VMEM((_TPW,), jnp.int32),
            pltpu.VMEM((_TPW, _D), jnp.float32),
            pltpu.SemaphoreType.DMA,
            pltpu.SemaphoreType.DMA,
        ],
    )
    def _dispatch(x_hbm, pos0_hbm, pos1_hbm, xs_hbm, idx0_v, idx1_v, rows_v,
                  sem0, sem1):
        wid = lax.axis_index("s") * 2 + lax.axis_index("c")
        base = wid * _TPW
        pltpu.sync_copy(x_hbm.at[pl.ds(base, _TPW)], rows_v)
        pltpu.sync_copy(pos0_hbm.at[pl.ds(base, _TPW)], idx0_v)
        pltpu.sync_copy(pos1_hbm.at[pl.ds(base, _TPW)], idx1_v)
        c0 = pltpu.async_copy(rows_v, xs_hbm.at[idx0_v], sem0)
        c1 = pltpu.async_copy(rows_v, xs_hbm.at[idx1_v], sem1)
        c0.wait()
        c1.wait()

    return _dispatch


def _expert_kernel(meta_ref, xs_ref, w1a_ref, w1b_ref, b1a_ref, b1b_ref,
                   w2_ref, b2_ref, ys_ref):
    i = pl.program_id(0)

    @pl.when(i < meta_ref[0])
    def _compute():
        xt = xs_ref[...]
        h1 = jnp.dot(xt, w1a_ref[0], preferred_element_type=jnp.float32)
        h1 = h1 + b1a_ref[0]
        h2 = jnp.dot(xt, w1b_ref[0], preferred_element_type=jnp.float32)
        h2 = h2 + b1b_ref[0]
        g = h1 * _gelu(h2)
        out = jnp.dot(g, w2_ref[0], preferred_element_type=jnp.float32)
        ys_ref[...] = out + b2_ref[0]


@functools.cache
def _get_combine():
    mesh = plsc.VectorSubcoreMesh(core_axis_name="c", subcore_axis_name="s")

    @functools.partial(
        pl.kernel,
        mesh=mesh,
        out_type=jax.ShapeDtypeStruct((_S, _D), jnp.float32),
        scratch_types=[
            pltpu.VMEM((_TPW,), jnp.int32),
            pltpu.VMEM((_TPW,), jnp.int32),
            pltpu.VMEM((_TPW, _LANES), jnp.float32),
            pltpu.VMEM((_TPW, _LANES), jnp.float32),
            pltpu.VMEM((_TPW, _D), jnp.float32),
            pltpu.VMEM((_TPW, _D), jnp.float32),
            pltpu.SemaphoreType.DMA,
            pltpu.SemaphoreType.DMA,
        ],
    )
    def _combine(ys_hbm, pos0_hbm, pos1_hbm, w0_hbm, w1_hbm, out_hbm,
                 idx0_v, idx1_v, w0_v, w1_v, rows0_v, rows1_v, sem0, sem1):
        wid = lax.axis_index("s") * 2 + lax.axis_index("c")
        base = wid * _TPW
        pltpu.sync_copy(pos0_hbm.at[pl.ds(base, _TPW)], idx0_v)
        pltpu.sync_copy(pos1_hbm.at[pl.ds(base, _TPW)], idx1_v)
        c0 = pltpu.async_copy(ys_hbm.at[idx0_v], rows0_v, sem0)
        c1 = pltpu.async_copy(ys_hbm.at[idx1_v], rows1_v, sem1)
        pltpu.sync_copy(w0_hbm.at[pl.ds(base, _TPW)], w0_v)
        pltpu.sync_copy(w1_hbm.at[pl.ds(base, _TPW)], w1_v)
        c0.wait()
        c1.wait()

        def body(t, carry):
            wv0 = w0_v[t, pl.ds(0, _LANES)]
            wv1 = w1_v[t, pl.ds(0, _LANES)]
            for cc in range(_D // _LANES):
                a = rows0_v[t, pl.ds(cc * _LANES, _LANES)]
                b = rows1_v[t, pl.ds(cc * _LANES, _LANES)]
                rows0_v[t, pl.ds(cc * _LANES, _LANES)] = wv0 * a + wv1 * b
            return carry

        lax.fori_loop(0, _TPW, body, 0)
        pltpu.sync_copy(rows0_v, out_hbm.at[pl.ds(base, _TPW)])

    return _combine


@jax.jit
def kernel(x, r_w1, r_b1, ln_scale, ln_bias, r_w2, r_b2, ew1, eb1, ew2, eb2,
           expert_priors):
    del expert_priors
    b, s, d = x.shape
    x2d = x.reshape(s, d)

    pos0, pos1, w0, w1, meta = pl.pallas_call(
        _router_kernel,
        out_shape=[
            jax.ShapeDtypeStruct((s, 1), jnp.int32),
            jax.ShapeDtypeStruct((s, 1), jnp.int32),
            jax.ShapeDtypeStruct((s, _LANES), jnp.float32),
            jax.ShapeDtypeStruct((s, _LANES), jnp.float32),
            jax.ShapeDtypeStruct((1, 64), jnp.int32),
        ],
    )(x2d, r_w1, r_b1.reshape(1, -1), ln_scale.reshape(1, -1),
      ln_bias.reshape(1, -1), r_w2, r_b2.reshape(1, -1))

    pos0f = pos0.reshape(s)
    pos1f = pos1.reshape(s)
    meta64 = meta.reshape(64)

    xs = _get_dispatch()(x2d, pos0f, pos1f)

    eb1_3d = eb1.reshape(_E, 1, 2 * _F)
    eb2_3d = eb2.reshape(_E, 1, _D)
    grid_spec = pltpu.PrefetchScalarGridSpec(
        num_scalar_prefetch=1,
        grid=(_TMAX,),
        in_specs=[
            pl.BlockSpec((_TILE, _D), lambda i, m: (i, 0)),
            pl.BlockSpec((1, _D, _F), lambda i, m: (m[1 + i], 0, 0)),
            pl.BlockSpec((1, _D, _F), lambda i, m: (m[1 + i], 0, 1)),
            pl.BlockSpec((1, 1, _F), lambda i, m: (m[1 + i], 0, 0)),
            pl.BlockSpec((1, 1, _F), lambda i, m: (m[1 + i], 0, 1)),
            pl.BlockSpec((1, _F, _D), lambda i, m: (m[1 + i], 0, 0)),
            pl.BlockSpec((1, 1, _D), lambda i, m: (m[1 + i], 0, 0)),
        ],
        out_specs=pl.BlockSpec((_TILE, _D), lambda i, m: (i, 0)),
    )
    ys = pl.pallas_call(
        _expert_kernel,
        grid_spec=grid_spec,
        out_shape=jax.ShapeDtypeStruct((_CAP, _D), jnp.float32),
        compiler_params=pltpu.CompilerParams(
            dimension_semantics=("arbitrary",),
            vmem_limit_bytes=100 * 1024 * 1024,
        ),
    )(meta64, xs, ew1, ew1, eb1_3d, eb1_3d, ew2, eb2_3d)

    out = _get_combine()(ys, pos0f, pos1f, w0, w1)

    return (out.reshape(b, s, d), 0.0)


# speedup vs baseline: 1.8365x; 1.0038x over previous
import functools

import jax
import jax.numpy as jnp
from jax import lax
from jax.experimental import pallas as pl
from jax.experimental.pallas import tpu as pltpu
from jax.experimental.pallas import tpu_sc as plsc

_D = 768
_F = 3072
_E = 8
_S = 2048
_TILE = 256
_TMAX = 24
_CAP = _TMAX * _TILE
_FBLK = 1024
_NF = _F // _FBLK
_NW = 32
_TPW = _S // _NW
_LANES = 16


def _gelu(x):
    sqrt_2_pi = 0.7978845608028654
    coef = 0.044715
    x3 = x ** 3
    inner = sqrt_2_pi * (x + coef * x3)
    return 0.5 * x * (1.0 + jnp.tanh(inner))


def _router_kernel(x_ref, w1_ref, b1_ref, lns_ref, lnb_ref, w2_ref, b2_ref,
                   pos0_ref, pos1_ref, w0_ref, w1o_ref, meta_ref):
    x = x_ref[...]
    h = jnp.dot(x, w1_ref[...], preferred_element_type=jnp.float32)
    h = h + b1_ref[...]
    mean = jnp.mean(h, axis=-1, keepdims=True)
    var = jnp.mean(jnp.square(h - mean), axis=-1, keepdims=True)
    h = (h - mean) * lax.rsqrt(var + 1e-6) * lns_ref[...] + lnb_ref[...]
    h = _gelu(h)
    logits = jnp.dot(h, w2_ref[...], preferred_element_type=jnp.float32)
    logits = logits + b2_ref[...]
    lmax = jnp.max(logits, axis=-1, keepdims=True)
    ex = jnp.exp(logits - lmax)
    p = ex / jnp.sum(ex, axis=-1, keepdims=True)

    s, e = p.shape
    ei = lax.broadcasted_iota(jnp.int32, (s, e), 1)
    m1 = jnp.max(p, axis=-1, keepdims=True)
    i1 = jnp.min(jnp.where(p == m1, ei, e), axis=-1, keepdims=True)
    oh1 = ei == i1
    pm = jnp.where(oh1, -jnp.inf, p)
    m2 = jnp.max(pm, axis=-1, keepdims=True)
    i2 = jnp.min(jnp.where(pm == m2, ei, e), axis=-1, keepdims=True)
    oh2 = ei == i2
    denom = m1 + m2
    w0_ref[...] = jnp.broadcast_to(m1 / denom, (s, _LANES))
    w1o_ref[...] = jnp.broadcast_to(m2 / denom, (s, _LANES))

    m01 = jnp.where(oh1 | oh2, 1.0, 0.0)
    l1 = lax.broadcasted_iota(jnp.int32, (128, 128), 0)
    l2 = lax.broadcasted_iota(jnp.int32, (128, 128), 1)
    ltri = jnp.where(l2 <= l1, 1.0, 0.0)
    nchunk = s // 128
    parts = []
    lasts = []
    for c in range(nchunk):
        blk = m01[c * 128:(c + 1) * 128, :]
        cumb = jnp.dot(ltri, blk, preferred_element_type=jnp.float32)
        parts.append(cumb)
        lasts.append(cumb[127:128, :])
    last = jnp.concatenate(lasts, axis=0)
    k1 = lax.broadcasted_iota(jnp.int32, (nchunk, nchunk), 0)
    k2 = lax.broadcasted_iota(jnp.int32, (nchunk, nchunk), 1)
    lo = jnp.where(k2 < k1, 1.0, 0.0)
    offs = jnp.dot(lo, last, preferred_element_type=jnp.float32)
    cum = jnp.concatenate(
        [parts[c] + offs[c:c + 1, :] for c in range(nchunk)], axis=0)

    counts = cum[s - 1:s, :]
    pc = jnp.ceil(counts / _TILE) * _TILE
    e1i = lax.broadcasted_iota(jnp.int32, (e, e), 0)
    e2i = lax.broadcasted_iota(jnp.int32, (e, e), 1)
    slt = jnp.where(e1i < e2i, 1.0, 0.0)
    aoff = jnp.dot(pc, slt, preferred_element_type=jnp.float32)

    g1 = jnp.sum(jnp.where(oh1, aoff + cum, 0.0), axis=-1, keepdims=True) - 1.0
    g2 = jnp.sum(jnp.where(oh2, aoff + cum, 0.0), axis=-1, keepdims=True) - 1.0
    pos0_ref[...] = g1.astype(jnp.int32)
    pos1_ref[...] = g2.astype(jnp.int32)

    endt = (aoff + pc) / _TILE
    eye = e1i == e2i
    end_col = jnp.sum(jnp.where(eye, jnp.broadcast_to(endt, (e, e)), 0.0),
                      axis=-1, keepdims=True)
    ti = lax.broadcasted_iota(jnp.int32, (e, 64), 1).astype(jnp.float32) - 1.0
    full_before = jnp.sum(jnp.where(ti >= end_col, 1.0, 0.0), axis=0,
                          keepdims=True)
    etile = jnp.minimum(full_before, float(e - 1))
    ntile = jnp.sum(pc, axis=-1, keepdims=True) / _TILE
    c64 = lax.broadcasted_iota(jnp.int32, (1, 64), 1)
    meta_ref[...] = jnp.where(c64 == 0, ntile, etile).astype(jnp.int32)


@functools.cache
def _get_dispatch():
    mesh = plsc.VectorSubcoreMesh(core_axis_name="c", subcore_axis_name="s")

    @functools.partial(
        pl.kernel,
        mesh=mesh,
        out_type=jax.ShapeDtypeStruct((_CAP, _D), jnp.float32),
        scratch_types=[
            pltpu.VMEM((_TPW,), jnp.int32),
            pltpu.VMEM((_TPW,), jnp.int32),
            pltpu.VMEM((_TPW, _D), jnp.float32),
            pltpu.SemaphoreType.DMA,
            pltpu.SemaphoreType.DMA,
        ],
    )
    def _dispatch(x_hbm, pos0_hbm, pos1_hbm, xs_hbm, idx0_v, idx1_v, rows_v,
                  sem0, sem1):
        wid = lax.axis_index("s") * 2 + lax.axis_index("c")
        base = wid * _TPW
        pltpu.sync_copy(x_hbm.at[pl.ds(base, _TPW)], rows_v)
        pltpu.sync_copy(pos0_hbm.at[pl.ds(base, _TPW)], idx0_v)
        pltpu.sync_copy(pos1_hbm.at[pl.ds(base, _TPW)], idx1_v)
        c0 = pltpu.async_copy(rows_v, xs_hbm.at[idx0_v], sem0)
        c1 = pltpu.async_copy(rows_v, xs_hbm.at[idx1_v], sem1)
        c0.wait()
        c1.wait()

    return _dispatch


def _expert_kernel(meta_ref, xs_ref, w1_ref, b1_ref, w2_ref, b2_ref, ys_ref):
    i = pl.program_id(0)

    @pl.when(i < meta_ref[0])
    def _compute():
        xt = xs_ref[...]
        h12 = jnp.dot(xt, w1_ref[0], preferred_element_type=jnp.float32)
        h12 = h12 + b1_ref[0]
        h1 = h12[:, :_F]
        h2 = h12[:, _F:]
        g = h1 * _gelu(h2)
        out = jnp.dot(g, w2_ref[0], preferred_element_type=jnp.float32)
        ys_ref[...] = out + b2_ref[0]


@functools.cache
def _get_combine():
    mesh = plsc.VectorSubcoreMesh(core_axis_name="c", subcore_axis_name="s")

    @functools.partial(
        pl.kernel,
        mesh=mesh,
        out_type=jax.ShapeDtypeStruct((_S, _D), jnp.float32),
        scratch_types=[
            pltpu.VMEM((_TPW,), jnp.int32),
            pltpu.VMEM((_TPW,), jnp.int32),
            pltpu.VMEM((_TPW, _LANES), jnp.float32),
            pltpu.VMEM((_TPW, _LANES), jnp.float32),
            pltpu.VMEM((_TPW, _D), jnp.float32),
            pltpu.VMEM((_TPW, _D), jnp.float32),
            pltpu.SemaphoreType.DMA,
            pltpu.SemaphoreType.DMA,
        ],
    )
    def _combine(ys_hbm, pos0_hbm, pos1_hbm, w0_hbm, w1_hbm, out_hbm,
                 idx0_v, idx1_v, w0_v, w1_v, rows0_v, rows1_v, sem0, sem1):
        wid = lax.axis_index("s") * 2 + lax.axis_index("c")
        base = wid * _TPW
        pltpu.sync_copy(pos0_hbm.at[pl.ds(base, _TPW)], idx0_v)
        pltpu.sync_copy(pos1_hbm.at[pl.ds(base, _TPW)], idx1_v)
        c0 = pltpu.async_copy(ys_hbm.at[idx0_v], rows0_v, sem0)
        c1 = pltpu.async_copy(ys_hbm.at[idx1_v], rows1_v, sem1)
        pltpu.sync_copy(w0_hbm.at[pl.ds(base, _TPW)], w0_v)
        pltpu.sync_copy(w1_hbm.at[pl.ds(base, _TPW)], w1_v)
        c0.wait()
        c1.wait()

        def body(t, carry):
            wv0 = w0_v[t, pl.ds(0, _LANES)]
            wv1 = w1_v[t, pl.ds(0, _LANES)]
            for cc in range(_D // _LANES):
                a = rows0_v[t, pl.ds(cc * _LANES, _LANES)]
                b = rows1_v[t, pl.ds(cc * _LANES, _LANES)]
                rows0_v[t, pl.ds(cc * _LANES, _LANES)] = wv0 * a + wv1 * b
            return carry

        lax.fori_loop(0, _TPW, body, 0)
        pltpu.sync_copy(rows0_v, out_hbm.at[pl.ds(base, _TPW)])

    return _combine


@jax.jit
def kernel(x, r_w1, r_b1, ln_scale, ln_bias, r_w2, r_b2, ew1, eb1, ew2, eb2,
           expert_priors):
    del expert_priors
    b, s, d = x.shape
    x2d = x.reshape(s, d)

    pos0, pos1, w0, w1, meta = pl.pallas_call(
        _router_kernel,
        out_shape=[
            jax.ShapeDtypeStruct((s, 1), jnp.int32),
            jax.ShapeDtypeStruct((s, 1), jnp.int32),
            jax.ShapeDtypeStruct((s, _LANES), jnp.float32),
            jax.ShapeDtypeStruct((s, _LANES), jnp.float32),
            jax.ShapeDtypeStruct((1, 64), jnp.int32),
        ],
    )(x2d, r_w1, r_b1.reshape(1, -1), ln_scale.reshape(1, -1),
      ln_bias.reshape(1, -1), r_w2, r_b2.reshape(1, -1))

    pos0f = pos0.reshape(s)
    pos1f = pos1.reshape(s)
    meta64 = meta.reshape(64)

    xs = _get_dispatch()(x2d, pos0f, pos1f)

    eb1_3d = eb1.reshape(_E, 1, 2 * _F)
    eb2_3d = eb2.reshape(_E, 1, _D)
    grid_spec = pltpu.PrefetchScalarGridSpec(
        num_scalar_prefetch=1,
        grid=(_TMAX,),
        in_specs=[
            pl.BlockSpec((_TILE, _D), lambda i, m: (i, 0)),
            pl.BlockSpec((1, _D, 2 * _F), lambda i, m: (m[1 + i], 0, 0)),
            pl.BlockSpec((1, 1, 2 * _F), lambda i, m: (m[1 + i], 0, 0)),
            pl.BlockSpec((1, _F, _D), lambda i, m: (m[1 + i], 0, 0)),
            pl.BlockSpec((1, 1, _D), lambda i, m: (m[1 + i], 0, 0)),
        ],
        out_specs=pl.BlockSpec((_TILE, _D), lambda i, m: (i, 0)),
    )
    ys = pl.pallas_call(
        _expert_kernel,
        grid_spec=grid_spec,
        out_shape=jax.ShapeDtypeStruct((_CAP, _D), jnp.float32),
        compiler_params=pltpu.CompilerParams(
            dimension_semantics=("arbitrary",),
            vmem_limit_bytes=100 * 1024 * 1024,
        ),
    )(meta64, xs, ew1, eb1_3d, ew2, eb2_3d)

    out = _get_combine()(ys, pos0f, pos1f, w0, w1)

    return (out.reshape(b, s, d), 0.0)
